# baseline (speedup 1.0000x reference)
import functools

import jax
import jax.numpy as jnp
from jax import lax
from jax.experimental import pallas as pl
from jax.experimental.pallas import tpu as pltpu
from jax.experimental.pallas import tpu_sc as plsc

NC = 2
NS = 16
LANES = 16
NW = NC * NS


_SC_CHUNK = 512


def _sc_coef_body(t_len, n_b, t_hbm, la_hbm, l1ma_hbm,
                  lac_hbm, l1mac_hbm, coef_hbm, t_v, la_v, l1ma_v,
                  lac_v, l1mac_v, coef_v, sem):
    wid = lax.axis_index("s") * NC + lax.axis_index("c")

    @pl.when(wid < n_b // LANES)
    def _():
        h0 = pltpu.async_copy(t_hbm, t_v, sem)
        h1 = pltpu.async_copy(la_hbm, la_v.at[pl.ds(0, t_len)], sem)
        h2 = pltpu.async_copy(l1ma_hbm, l1ma_v.at[pl.ds(0, t_len)], sem)
        h3 = pltpu.async_copy(lac_hbm, lac_v.at[pl.ds(0, t_len)], sem)
        h4 = pltpu.async_copy(l1mac_hbm, l1mac_v.at[pl.ds(0, t_len)], sem)
        h0.wait()
        h1.wait()
        h2.wait()
        h3.wait()
        h4.wait()

        zero = jnp.zeros((LANES,), jnp.int32)
        iota = lax.iota(jnp.int32, LANES)
        tv = t_v[pl.ds(wid * LANES, LANES)]
        tm1 = jnp.maximum(tv - 1, 0)
        a = plsc.load_gather(lac_v, [tm1])
        c = plsc.load_gather(l1mac_v, [tm1])
        d = plsc.load_gather(la_v, [tv])
        e = plsc.load_gather(l1ma_v, [tv])
        plsc.store_scatter(coef_v, [iota, zero], a - c)
        plsc.store_scatter(coef_v, [iota, zero + 1], d - e)
        pltpu.sync_copy(coef_v, coef_hbm.at[pl.ds(wid * LANES, LANES)])


def _sc_coef(t, la, l1ma, lac, l1mac):
    b = t.shape[0]
    t_len = la.shape[0]
    t_pad = ((t_len + LANES - 1) // LANES) * LANES
    mesh = plsc.VectorSubcoreMesh(core_axis_name="c", subcore_axis_name="s",
                                  num_cores=NC, num_subcores=NS)
    body = functools.partial(_sc_coef_body, t_len, b)
    return pl.kernel(
        body,
        out_type=jax.ShapeDtypeStruct((b, 128), jnp.float32),
        mesh=mesh,
        scratch_types=[
            pltpu.VMEM((b,), jnp.int32),
            pltpu.VMEM((t_pad,), jnp.float32),
            pltpu.VMEM((t_pad,), jnp.float32),
            pltpu.VMEM((t_pad,), jnp.float32),
            pltpu.VMEM((t_pad,), jnp.float32),
            pltpu.VMEM((LANES, 128), jnp.float32),
            pltpu.SemaphoreType.DMA,
        ],
        compiler_params=pltpu.CompilerParams(needs_layout_passes=False),
    )(t, la, l1ma, lac, l1mac)


def _tc_body(v0_ref, vt_ref, batch_ref, cb_ref, prior_ref, out_ref):
    p = prior_ref[...]
    block, n_b = v0_ref.shape[0], cb_ref.shape[0]
    bt = batch_ref[0]
    oht = (bt == lax.broadcasted_iota(jnp.int32, (n_b, block), 0))
    coef = jax.lax.dot_general(oht.astype(jnp.float32), cb_ref[...],
                               (((0,), (0,)), ((), ())),
                               preferred_element_type=jnp.float32)
    a = coef[:, 0:1]
    d = coef[:, 1:2]
    x1 = v0_ref[...] + a
    x2 = vt_ref[...] + d
    e1 = jnp.exp(-jnp.abs(x1 - p))
    e2 = jnp.exp(-jnp.abs(x2 - p))
    w = jnp.maximum(x1, p) + jnp.maximum(x2, p)
    u = w + jnp.log((1.0 + e1) * (1.0 + e2))
    m = jnp.max(u, axis=1, keepdims=True)
    lse = m + jnp.log(jnp.sum(jnp.exp(u - m), axis=1, keepdims=True))
    out_ref[...] = u - lse


def _tc_dense(log_v0, log_vt, batch, cb, prior, block=1024):
    n, k = log_v0.shape
    n_b = cb.shape[0]
    grid = (n // block,)
    batch3 = batch.reshape(n // block, 1, block)
    return pl.pallas_call(
        _tc_body,
        grid=grid,
        in_specs=[
            pl.BlockSpec((block, k), lambda i: (i, 0)),
            pl.BlockSpec((block, k), lambda i: (i, 0)),
            pl.BlockSpec((1, 1, block), lambda i: (i, 0, 0)),
            pl.BlockSpec((n_b, 128), lambda i: (0, 0)),
            pl.BlockSpec((1, k), lambda i: (0, 0)),
        ],
        out_specs=pl.BlockSpec((block, k), lambda i: (i, 0)),
        out_shape=jax.ShapeDtypeStruct((n, k), jnp.float32),
        compiler_params=pltpu.CompilerParams(
            dimension_semantics=("parallel",),
        ),
    )(log_v0, log_vt, batch3, cb, prior)


def kernel(log_v0, log_vt, t, batch, log_alphas_v, log_one_minus_alphas_v,
           log_alphas_cumprod_v, log_one_minus_alphas_cumprod_v, prior_probs):
    cb = _sc_coef(t, log_alphas_v, log_one_minus_alphas_v,
                  log_alphas_cumprod_v, log_one_minus_alphas_cumprod_v)
    return _tc_dense(log_v0, log_vt, batch, cb, prior_probs)


# speedup vs baseline: 1.0592x; 1.0592x over previous
import functools

import jax
import jax.numpy as jnp
from jax import lax
from jax.experimental import pallas as pl
from jax.experimental.pallas import tpu as pltpu
from jax.experimental.pallas import tpu_sc as plsc

NC = 2
NS = 16
LANES = 16
NW = NC * NS


_SC_CHUNK = 512


def _sc_coef_body(t_len, n_b, t_hbm, la_hbm, l1ma_hbm,
                  lac_hbm, l1mac_hbm, coef_hbm, t_v, la_v, l1ma_v,
                  lac_v, l1mac_v, coef_v, sem):
    wid = lax.axis_index("s") * NC + lax.axis_index("c")

    @pl.when(wid < n_b // LANES)
    def _():
        h0 = pltpu.async_copy(t_hbm, t_v, sem)
        h1 = pltpu.async_copy(la_hbm, la_v.at[pl.ds(0, t_len)], sem)
        h2 = pltpu.async_copy(l1ma_hbm, l1ma_v.at[pl.ds(0, t_len)], sem)
        h3 = pltpu.async_copy(lac_hbm, lac_v.at[pl.ds(0, t_len)], sem)
        h4 = pltpu.async_copy(l1mac_hbm, l1mac_v.at[pl.ds(0, t_len)], sem)
        h0.wait()
        h1.wait()
        h2.wait()
        h3.wait()
        h4.wait()

        zero = jnp.zeros((LANES,), jnp.int32)
        iota = lax.iota(jnp.int32, LANES)
        tv = t_v[pl.ds(wid * LANES, LANES)]
        tm1 = jnp.maximum(tv - 1, 0)
        a = plsc.load_gather(lac_v, [tm1])
        c = plsc.load_gather(l1mac_v, [tm1])
        d = plsc.load_gather(la_v, [tv])
        e = plsc.load_gather(l1ma_v, [tv])
        plsc.store_scatter(coef_v, [iota, zero], a - c)
        plsc.store_scatter(coef_v, [iota, zero + 1], d - e)
        pltpu.sync_copy(coef_v, coef_hbm.at[pl.ds(wid * LANES, LANES)])


def _sc_coef(t, la, l1ma, lac, l1mac):
    b = t.shape[0]
    t_len = la.shape[0]
    t_pad = ((t_len + LANES - 1) // LANES) * LANES
    mesh = plsc.VectorSubcoreMesh(core_axis_name="c", subcore_axis_name="s",
                                  num_cores=NC, num_subcores=NS)
    body = functools.partial(_sc_coef_body, t_len, b)
    return pl.kernel(
        body,
        out_type=jax.ShapeDtypeStruct((b, 128), jnp.float32),
        mesh=mesh,
        scratch_types=[
            pltpu.VMEM((b,), jnp.int32),
            pltpu.VMEM((t_pad,), jnp.float32),
            pltpu.VMEM((t_pad,), jnp.float32),
            pltpu.VMEM((t_pad,), jnp.float32),
            pltpu.VMEM((t_pad,), jnp.float32),
            pltpu.VMEM((LANES, 128), jnp.float32),
            pltpu.SemaphoreType.DMA,
        ],
        compiler_params=pltpu.CompilerParams(needs_layout_passes=False),
    )(t, la, l1ma, lac, l1mac)


def _tc_body(v0_ref, vt_ref, batch_ref, cb_ref, prior_ref, out_ref):
    p = prior_ref[...]
    block, n_b = v0_ref.shape[0], cb_ref.shape[0]
    i = pl.program_id(0)
    bt = batch_ref[0, 0:1, pl.ds(i * block, block)]
    oht = (bt == lax.broadcasted_iota(jnp.int32, (n_b, block), 0))
    coef = jax.lax.dot_general(oht.astype(jnp.float32), cb_ref[...],
                               (((0,), (0,)), ((), ())),
                               preferred_element_type=jnp.float32)
    a = coef[:, 0:1]
    d = coef[:, 1:2]
    x1 = v0_ref[...] + a
    x2 = vt_ref[...] + d
    e1 = jnp.exp(-jnp.abs(x1 - p))
    e2 = jnp.exp(-jnp.abs(x2 - p))
    w = jnp.maximum(x1, p) + jnp.maximum(x2, p)
    u = w + jnp.log((1.0 + e1) * (1.0 + e2))
    m = jnp.max(u, axis=1, keepdims=True)
    lse = m + jnp.log(jnp.sum(jnp.exp(u - m), axis=1, keepdims=True))
    out_ref[...] = u - lse


def _tc_dense(log_v0, log_vt, batch, cb, prior, block=2048):
    n, k = log_v0.shape
    n_b = cb.shape[0]
    grid = (n // block,)
    batch3 = batch.reshape(1, 1, n)
    return pl.pallas_call(
        _tc_body,
        grid=grid,
        in_specs=[
            pl.BlockSpec((block, k), lambda i: (i, 0)),
            pl.BlockSpec((block, k), lambda i: (i, 0)),
            pl.BlockSpec((1, 1, n), lambda i: (0, 0, 0)),
            pl.BlockSpec((n_b, 128), lambda i: (0, 0)),
            pl.BlockSpec((1, k), lambda i: (0, 0)),
        ],
        out_specs=pl.BlockSpec((block, k), lambda i: (i, 0)),
        out_shape=jax.ShapeDtypeStruct((n, k), jnp.float32),
        compiler_params=pltpu.CompilerParams(
            dimension_semantics=("parallel",),
        ),
    )(log_v0, log_vt, batch3, cb, prior)


def kernel(log_v0, log_vt, t, batch, log_alphas_v, log_one_minus_alphas_v,
           log_alphas_cumprod_v, log_one_minus_alphas_cumprod_v, prior_probs):
    cb = _sc_coef(t, log_alphas_v, log_one_minus_alphas_v,
                  log_alphas_cumprod_v, log_one_minus_alphas_cumprod_v)
    return _tc_dense(log_v0, log_vt, batch, cb, prior_probs)
